# baseline (speedup 1.0000x reference)
import functools

import jax
import jax.numpy as jnp
from jax import lax
from jax.experimental import pallas as pl
from jax.experimental.pallas import tpu as pltpu
from jax.experimental.pallas import tpu_sc as plsc

N_EMBED = 4096
NUM_EXPERTS = 64
TOP_K = 8
N_TOKENS = 8192

TOKEN_BLOCK = 1024

NUM_WORKERS = 32
TPW = N_TOKENS // NUM_WORKERS
NGROUPS = TPW // 16


def _logits_kernel(x_ref, wr_ref, wn_ref, br_ref, bn_ref, eps_ref, noisy_t_ref, wcat_ref):
    @pl.when(pl.program_id(0) == 0)
    def _pack_weights():
        wcat_ref[:NUM_EXPERTS, :] = wr_ref[...].astype(jnp.bfloat16)
        wcat_ref[NUM_EXPERTS:, :] = wn_ref[...].astype(jnp.bfloat16)

    x = x_ref[...].astype(jnp.bfloat16)
    logits_cat = jax.lax.dot_general(
        x,
        wcat_ref[...],
        dimension_numbers=(((1,), (1,)), ((), ())),
        preferred_element_type=jnp.float32,
    )
    logits = logits_cat[:, :NUM_EXPERTS] + br_ref[...]
    noise_logits = logits_cat[:, NUM_EXPERTS:] + bn_ref[...]
    noisy = logits + eps_ref[...] * jax.nn.softplus(noise_logits)
    noisy_t_ref[...] = noisy.T


def _topk_sc_body(noisy_hbm, router_hbm, idx_hbm, vals, probs, idxb, mkb, sem):
    wid = lax.axis_index("s") * 2 + lax.axis_index("c")
    base = wid * TPW
    in_copies = [
        pltpu.async_copy(
            noisy_hbm.at[e, pl.ds(base, TPW)], vals.at[pl.ds(e * TPW, TPW)], sem
        )
        for e in range(NUM_EXPERTS)
    ]
    for c in in_copies:
        c.wait()

    neg_inf = jnp.float32(-jnp.inf)

    def group_body(g, carry):
        col0 = g * 16

        def k_body(k, mi):
            m_prev, idx_prev = mi
            m = jnp.full((16,), neg_inf, jnp.float32)
            for e in range(NUM_EXPERTS):
                v = vals[pl.ds(e * TPW + col0, 16)]
                elig = (v < m_prev) | ((v == m_prev) & (jnp.int32(e) > idx_prev))
                m = jnp.maximum(m, jnp.where(elig, v, neg_inf))
            idx = jnp.full((16,), NUM_EXPERTS, jnp.int32)
            for e in range(NUM_EXPERTS):
                v = vals[pl.ds(e * TPW + col0, 16)]
                elig = (v < m_prev) | ((v == m_prev) & (jnp.int32(e) > idx_prev))
                hit = elig & (v == m)
                idx = jnp.minimum(idx, jnp.where(hit, jnp.int32(e), jnp.int32(NUM_EXPERTS)))
            idxb[pl.ds(k * TPW + col0, 16)] = idx
            mkb[pl.ds(k * 16, 16)] = m
            return (m, idx)

        m8, idx8 = lax.fori_loop(
            0,
            TOP_K,
            k_body,
            (jnp.full((16,), jnp.inf, jnp.float32), jnp.full((16,), -1, jnp.int32)),
        )

        top1 = mkb[pl.ds(0, 16)]
        denom = jnp.zeros((16,), jnp.float32)
        for k in range(TOP_K):
            denom = denom + jnp.exp(mkb[pl.ds(k * 16, 16)] - top1)
        inv = 1.0 / denom
        for e in range(NUM_EXPERTS):
            v = vals[pl.ds(e * TPW + col0, 16)]
            sel = (v > m8) | ((v == m8) & (jnp.int32(e) <= idx8))
            probs[pl.ds(e * TPW + col0, 16)] = jnp.where(
                sel, jnp.exp(v - top1) * inv, 0.0
            )
        return carry

    lax.fori_loop(0, NGROUPS, group_body, 0)

    out_copies = [
        pltpu.async_copy(
            probs.at[pl.ds(e * TPW, TPW)], router_hbm.at[e, pl.ds(base, TPW)], sem
        )
        for e in range(NUM_EXPERTS)
    ] + [
        pltpu.async_copy(
            idxb.at[pl.ds(k * TPW, TPW)], idx_hbm.at[k, pl.ds(base, TPW)], sem
        )
        for k in range(TOP_K)
    ]
    for c in out_copies:
        c.wait()


def kernel(mh_output, W_route, b_route, W_noise, b_noise, noise_eps):
    n_tokens = mh_output.shape[0]
    grid = (n_tokens // TOKEN_BLOCK,)

    noisy_t = pl.pallas_call(
        _logits_kernel,
        grid=grid,
        in_specs=[
            pl.BlockSpec((TOKEN_BLOCK, N_EMBED), lambda i: (i, 0)),
            pl.BlockSpec((NUM_EXPERTS, N_EMBED), lambda i: (0, 0)),
            pl.BlockSpec((NUM_EXPERTS, N_EMBED), lambda i: (0, 0)),
            pl.BlockSpec((1, NUM_EXPERTS), lambda i: (0, 0)),
            pl.BlockSpec((1, NUM_EXPERTS), lambda i: (0, 0)),
            pl.BlockSpec((TOKEN_BLOCK, NUM_EXPERTS), lambda i: (i, 0)),
        ],
        out_specs=pl.BlockSpec((NUM_EXPERTS, TOKEN_BLOCK), lambda i: (0, i)),
        out_shape=jax.ShapeDtypeStruct((NUM_EXPERTS, n_tokens), jnp.float32),
        scratch_shapes=[pltpu.VMEM((2 * NUM_EXPERTS, N_EMBED), jnp.bfloat16)],
        compiler_params=pltpu.CompilerParams(
            dimension_semantics=("parallel",),
        ),
    )(mh_output, W_route, W_noise, b_route[None, :], b_noise[None, :], noise_eps)

    mesh = plsc.VectorSubcoreMesh(core_axis_name="c", subcore_axis_name="s")
    router_t, idx_t = functools.partial(
        pl.kernel,
        mesh=mesh,
        out_type=[
            jax.ShapeDtypeStruct((NUM_EXPERTS, n_tokens), jnp.float32),
            jax.ShapeDtypeStruct((TOP_K, n_tokens), jnp.int32),
        ],
        scratch_types=[
            pltpu.VMEM((NUM_EXPERTS * TPW,), jnp.float32),
            pltpu.VMEM((NUM_EXPERTS * TPW,), jnp.float32),
            pltpu.VMEM((TOP_K * TPW,), jnp.int32),
            pltpu.VMEM((TOP_K * 16,), jnp.float32),
            pltpu.SemaphoreType.DMA,
        ],
    )(_topk_sc_body)(noisy_t)

    return (router_t.T, idx_t.T)


# speedup vs baseline: 1.7941x; 1.7941x over previous
import functools

import jax
import jax.numpy as jnp
from jax.experimental import pallas as pl
from jax.experimental.pallas import tpu as pltpu

N_EMBED = 4096
NUM_EXPERTS = 64
TOP_K = 8
N_TOKENS = 8192

TOKEN_BLOCK = 1024


KSPLIT = 2
K_BLOCK = N_EMBED // KSPLIT


def _router_kernel(
    x_ref, wr_ref, wn_ref, br_ref, bn_ref, eps_ref, router_ref, idx_ref, wcat_ref, acc_ref
):
    kstep = pl.program_id(1)

    @pl.when((pl.program_id(0) == 0) & (kstep == 0))
    def _pack_weights():
        wcat_ref[:NUM_EXPERTS, :] = wr_ref[...].astype(jnp.bfloat16)
        wcat_ref[NUM_EXPERTS:, :] = wn_ref[...].astype(jnp.bfloat16)

    x = x_ref[...].astype(jnp.bfloat16)
    partial_dot = jax.lax.dot_general(
        x,
        wcat_ref[:, pl.ds(kstep * K_BLOCK, K_BLOCK)],
        dimension_numbers=(((1,), (1,)), ((), ())),
        preferred_element_type=jnp.float32,
    )

    @pl.when(kstep == 0)
    def _init_acc():
        acc_ref[...] = partial_dot

    @pl.when(kstep > 0)
    def _add_acc():
        acc_ref[...] += partial_dot

    @pl.when(kstep == KSPLIT - 1)
    def _epilogue():
        _finish(acc_ref, br_ref, bn_ref, eps_ref, router_ref, idx_ref)


def _finish(acc_ref, br_ref, bn_ref, eps_ref, router_ref, idx_ref):
    logits_cat = acc_ref[...]
    logits = logits_cat[:, :NUM_EXPERTS] + br_ref[...]
    noise_logits = logits_cat[:, NUM_EXPERTS:] + bn_ref[...]
    noisy = logits + eps_ref[...] * jax.nn.softplus(noise_logits)

    noisy_t = noisy.T
    rowf = jax.lax.broadcasted_iota(jnp.int32, noisy_t.shape, 0).astype(jnp.float32)
    vals = noisy_t
    neg_inf = jnp.float32(-jnp.inf)
    top1 = None
    idx_rows = []
    for k in range(TOP_K):
        m = jnp.max(vals, axis=0, keepdims=True)
        if k == 0:
            top1 = m
        idx = jnp.min(
            jnp.where(vals == m, rowf, jnp.float32(NUM_EXPERTS)),
            axis=0,
            keepdims=True,
        )
        idx_rows.append(idx)
        vals = jnp.where(rowf == idx, neg_inf, vals)

    idx_t = jnp.concatenate(idx_rows, axis=0)
    idx_ref[...] = idx_t.T.astype(jnp.int32)

    selected = vals == neg_inf
    e = jnp.where(selected, jnp.exp(noisy_t - top1), 0.0)
    denom = jnp.sum(e, axis=0, keepdims=True)
    router_ref[...] = (e / denom).T


def kernel(mh_output, W_route, b_route, W_noise, b_noise, noise_eps):
    n_tokens = mh_output.shape[0]
    grid = (n_tokens // TOKEN_BLOCK, KSPLIT)

    router_out, idx_out = pl.pallas_call(
        _router_kernel,
        grid=grid,
        in_specs=[
            pl.BlockSpec((TOKEN_BLOCK, K_BLOCK), lambda i, k: (i, k)),
            pl.BlockSpec((NUM_EXPERTS, N_EMBED), lambda i, k: (0, 0)),
            pl.BlockSpec((NUM_EXPERTS, N_EMBED), lambda i, k: (0, 0)),
            pl.BlockSpec((1, NUM_EXPERTS), lambda i, k: (0, 0)),
            pl.BlockSpec((1, NUM_EXPERTS), lambda i, k: (0, 0)),
            pl.BlockSpec((TOKEN_BLOCK, NUM_EXPERTS), lambda i, k: (i, 0)),
        ],
        out_specs=[
            pl.BlockSpec((TOKEN_BLOCK, NUM_EXPERTS), lambda i, k: (i, 0)),
            pl.BlockSpec((TOKEN_BLOCK, TOP_K), lambda i, k: (i, 0)),
        ],
        out_shape=[
            jax.ShapeDtypeStruct((n_tokens, NUM_EXPERTS), jnp.float32),
            jax.ShapeDtypeStruct((n_tokens, TOP_K), jnp.int32),
        ],
        scratch_shapes=[
            pltpu.VMEM((2 * NUM_EXPERTS, N_EMBED), jnp.bfloat16),
            pltpu.VMEM((TOKEN_BLOCK, 2 * NUM_EXPERTS), jnp.float32),
        ],
        compiler_params=pltpu.CompilerParams(
            dimension_semantics=("parallel", "arbitrary"),
        ),
    )(mh_output, W_route, W_noise, b_route[None, :], b_noise[None, :], noise_eps)

    return (router_out, idx_out)


# speedup vs baseline: 2.0215x; 1.1268x over previous
import functools

import jax
import jax.numpy as jnp
from jax.experimental import pallas as pl
from jax.experimental.pallas import tpu as pltpu

N_EMBED = 4096
NUM_EXPERTS = 64
TOP_K = 8
N_TOKENS = 8192

TOKEN_BLOCK = 1024


def _router_kernel(
    x_ref, wr_ref, wn_ref, br_ref, bn_ref, eps_ref, router_ref, idx_ref, wcat_ref
):
    @pl.when(pl.program_id(0) == 0)
    def _pack_weights():
        wcat_ref[:NUM_EXPERTS, :] = wr_ref[...].astype(jnp.bfloat16)
        wcat_ref[NUM_EXPERTS:, :] = wn_ref[...].astype(jnp.bfloat16)

    x = x_ref[...].astype(jnp.bfloat16)
    logits_cat = jax.lax.dot_general(
        x,
        wcat_ref[...],
        dimension_numbers=(((1,), (1,)), ((), ())),
        preferred_element_type=jnp.float32,
    )

    logits = logits_cat[:, :NUM_EXPERTS] + br_ref[...]
    noise_logits = logits_cat[:, NUM_EXPERTS:] + bn_ref[...]
    noisy = logits + eps_ref[...] * jax.nn.softplus(noise_logits)

    noisy_t = noisy.T
    rowf = jax.lax.broadcasted_iota(jnp.int32, noisy_t.shape, 0).astype(jnp.float32)
    vals = noisy_t
    neg_inf = jnp.float32(-jnp.inf)
    top1 = None
    idx_rows = []
    for k in range(TOP_K):
        m = jnp.max(vals, axis=0, keepdims=True)
        if k == 0:
            top1 = m
        idx = jnp.min(
            jnp.where(vals == m, rowf, jnp.float32(NUM_EXPERTS)),
            axis=0,
            keepdims=True,
        )
        idx_rows.append(idx)
        vals = jnp.where(rowf == idx, neg_inf, vals)

    idx_t = jnp.concatenate(idx_rows, axis=0)
    idx_ref[...] = idx_t.T.astype(jnp.int32)

    selected = vals == neg_inf
    e = jnp.where(selected, jnp.exp(noisy_t - top1), 0.0)
    denom = jnp.sum(e, axis=0, keepdims=True)
    router_ref[...] = (e / denom).T


def kernel(mh_output, W_route, b_route, W_noise, b_noise, noise_eps):
    n_tokens = mh_output.shape[0]
    grid = (n_tokens // TOKEN_BLOCK,)

    router_out, idx_out = pl.pallas_call(
        _router_kernel,
        grid=grid,
        in_specs=[
            pl.BlockSpec((TOKEN_BLOCK, N_EMBED), lambda i: (i, 0)),
            pl.BlockSpec((NUM_EXPERTS, N_EMBED), lambda i: (0, 0)),
            pl.BlockSpec((NUM_EXPERTS, N_EMBED), lambda i: (0, 0)),
            pl.BlockSpec((1, NUM_EXPERTS), lambda i: (0, 0)),
            pl.BlockSpec((1, NUM_EXPERTS), lambda i: (0, 0)),
            pl.BlockSpec((TOKEN_BLOCK, NUM_EXPERTS), lambda i: (i, 0)),
        ],
        out_specs=[
            pl.BlockSpec((TOKEN_BLOCK, NUM_EXPERTS), lambda i: (i, 0)),
            pl.BlockSpec((TOKEN_BLOCK, TOP_K), lambda i: (i, 0)),
        ],
        out_shape=[
            jax.ShapeDtypeStruct((n_tokens, NUM_EXPERTS), jnp.float32),
            jax.ShapeDtypeStruct((n_tokens, TOP_K), jnp.int32),
        ],
        scratch_shapes=[pltpu.VMEM((2 * NUM_EXPERTS, N_EMBED), jnp.bfloat16)],
        compiler_params=pltpu.CompilerParams(
            dimension_semantics=("parallel",),
        ),
    )(mh_output, W_route, W_noise, b_route[None, :], b_noise[None, :], noise_eps)

    return (router_out, idx_out)
